# baseline (speedup 1.0000x reference)
import functools

import jax
import jax.numpy as jnp
from jax import lax
from jax.experimental import pallas as pl
from jax.experimental.pallas import tpu as pltpu
from jax.experimental.pallas import tpu_sc as plsc

Q = 4096
N = 8192
K = 10
R2 = 0.25 * 0.25
BQ = 256
KPAD = 16
PAD_ROW = N
TROWS = N + 8

SC_CORES = 2
SC_SUBCORES = 16
NW = SC_CORES * SC_SUBCORES
B_TOTAL = Q * K
B_PER_W = B_TOTAL // NW
CHUNK = 128
NCHUNK = B_PER_W // CHUNK


def _topk_body(q_ref, rt_ref, map_ref, gidx_ref):
    q = q_ref[...]
    rt = rt_ref[...]
    qb = q.astype(jnp.bfloat16)
    rb = rt.astype(jnp.bfloat16)
    cross = jnp.dot(qb, rb, preferred_element_type=jnp.float32)
    qn = ((q[:, 0:1] * q[:, 0:1] + q[:, 1:2] * q[:, 1:2])
          + q[:, 2:3] * q[:, 2:3])
    rn = ((rt[0:1, :] * rt[0:1, :] + rt[1:2, :] * rt[1:2, :])
          + rt[2:3, :] * rt[2:3, :])
    d2 = (qn + rn) - 2.0 * cross
    d2 = jnp.maximum(d2, 0.0)
    neg_inf = jnp.float32(-jnp.inf)
    score = jnp.where(d2 <= R2, -d2, neg_inf)
    iota = lax.broadcasted_iota(jnp.int32, score.shape, 1)

    def direct(v, i, k):
        cands = []
        for _ in range(k):
            m = jnp.max(v, axis=1, keepdims=True)
            ci = jnp.min(jnp.where(v == m, i, N), axis=1)
            cands.append((m[:, 0], ci))
            v = jnp.where(i == ci[:, None], neg_inf, v)
        return cands

    def extract(v, i, k, top):
        if k == 0:
            return []
        h = v.shape[1] // 2
        if k == 1 or h < 1024:
            return direct(v, i, k)
        a, b = v[:, :h], v[:, h:]
        ia, ib = i[:, :h], i[:, h:]
        if top:
            better = b > a
        else:
            better = (b > a) | ((b == a) & (ib < ia))
        w = jnp.where(better, b, a)
        iw = jnp.where(better, ib, ia)
        l = jnp.where(better, a, b)
        il = jnp.where(better, ia, ib)
        return (extract(w, iw, k, False)
                + extract(l, il, k // 2, False))

    cands = extract(score, iota, K, True)
    cv = jnp.stack([c[0] for c in cands], axis=1)
    ci_all = jnp.stack([c[1] for c in cands], axis=1)
    for k in range(K):
        m = jnp.max(cv, axis=1, keepdims=True)
        ci = jnp.min(jnp.where(cv == m, ci_all, N), axis=1)
        valid = m[:, 0] > neg_inf
        map_ref[k, :] = jnp.where(valid, ci, 0)
        gidx_ref[k, :] = jnp.where(valid, ci, PAD_ROW)
        cv = jnp.where(ci_all == ci[:, None], neg_inf, cv)


def _topk_tc(qpad, rt):
    grid = (Q // BQ,)
    return pl.pallas_call(
        _topk_body,
        grid=grid,
        in_specs=[
            pl.BlockSpec((BQ, 8), lambda i: (i, 0)),
            pl.BlockSpec((8, N), lambda i: (0, 0)),
        ],
        out_specs=[
            pl.BlockSpec((KPAD, BQ), lambda i: (0, i)),
            pl.BlockSpec((KPAD, BQ), lambda i: (0, i)),
        ],
        out_shape=[
            jax.ShapeDtypeStruct((KPAD, Q), jnp.int32),
            jax.ShapeDtypeStruct((KPAD, Q), jnp.int32),
        ],
        compiler_params=pltpu.CompilerParams(
            dimension_semantics=("arbitrary",),
        ),
    )(qpad, rt)


def _gather_body(table_hbm, idx_hbm, out_hbm, idx_v, rows_v, sem):
    wid = lax.axis_index("s") * SC_CORES + lax.axis_index("c")
    base = wid * B_PER_W
    pltpu.sync_copy(idx_hbm.at[wid], idx_v)
    for j in range(NCHUNK):
        pltpu.async_copy(
            table_hbm.at[idx_v.at[j]],
            rows_v.at[pl.ds(j * CHUNK, CHUNK)],
            sem,
        ).wait()
    pltpu.sync_copy(rows_v, out_hbm.at[pl.ds(base, B_PER_W)])


@functools.lru_cache(maxsize=1)
def _gather_sc():
    return pl.kernel(
        _gather_body,
        out_type=jax.ShapeDtypeStruct((B_TOTAL, 16), jnp.float32),
        mesh=plsc.VectorSubcoreMesh(
            core_axis_name="c", subcore_axis_name="s",
            num_cores=SC_CORES, num_subcores=SC_SUBCORES,
        ),
        scratch_types=[
            pltpu.VMEM((NCHUNK, CHUNK), jnp.int32),
            pltpu.VMEM((B_PER_W, 16), jnp.float32),
            pltpu.SemaphoreType.DMA,
        ],
        compiler_params=pltpu.CompilerParams(use_tc_tiling_on_sc=False),
    )


def kernel(x, p_grid):
    refs = x[0]
    q = jnp.reshape(p_grid, (Q, 3))
    qpad = jnp.pad(q, ((0, 0), (0, 5)))
    rt = jnp.pad(refs, ((0, 0), (0, 5))).T

    map16, gidx16 = _topk_tc(qpad, rt)
    mapping = map16[:K].T
    gidx3d = jnp.reshape(gidx16[:K].T, (NW, NCHUNK, CHUNK))

    table = jnp.zeros((TROWS, 16), jnp.float32).at[:N, :3].set(refs)
    rows = _gather_sc()(table, gidx3d)
    outputs = jnp.reshape(rows[:, :3], (1, Q, K, 3))
    return jnp.reshape(mapping, (1, Q, K)), outputs


# speedup vs baseline: 1.0252x; 1.0252x over previous
import functools

import jax
import jax.numpy as jnp
from jax import lax
from jax.experimental import pallas as pl
from jax.experimental.pallas import tpu as pltpu
from jax.experimental.pallas import tpu_sc as plsc

Q = 4096
N = 8192
K = 10
R2 = 0.25 * 0.25
BQ = 128
KPAD = 16
PAD_ROW = N
TROWS = N + 8

SC_CORES = 2
SC_SUBCORES = 16
NW = SC_CORES * SC_SUBCORES
B_TOTAL = Q * K
B_PER_W = B_TOTAL // NW
CHUNK = 128
NCHUNK = B_PER_W // CHUNK


def _topk_body(q_ref, rt_ref, map_ref, gidx_ref):
    q = q_ref[...]
    rt = rt_ref[...]
    qb = q.astype(jnp.bfloat16)
    rb = rt.astype(jnp.bfloat16)
    cross = jnp.dot(qb, rb, preferred_element_type=jnp.float32)
    qn = ((q[:, 0:1] * q[:, 0:1] + q[:, 1:2] * q[:, 1:2])
          + q[:, 2:3] * q[:, 2:3])
    rn = ((rt[0:1, :] * rt[0:1, :] + rt[1:2, :] * rt[1:2, :])
          + rt[2:3, :] * rt[2:3, :])
    d2 = (qn + rn) - 2.0 * cross
    d2 = jnp.maximum(d2, 0.0)
    neg_inf = jnp.float32(-jnp.inf)
    score = jnp.where(d2 <= R2, -d2, neg_inf)
    iota = lax.broadcasted_iota(jnp.int32, score.shape, 1)

    def direct(v, i, k):
        cands = []
        for _ in range(k):
            m = jnp.max(v, axis=1, keepdims=True)
            ci = jnp.min(jnp.where(v == m, i, N), axis=1)
            cands.append((m[:, 0], ci))
            v = jnp.where(i == ci[:, None], neg_inf, v)
        return cands

    def extract(v, i, k, top):
        if k == 0:
            return []
        h = v.shape[1] // 2
        if k == 1 or h < 512:
            return direct(v, i, k)
        a, b = v[:, :h], v[:, h:]
        ia, ib = i[:, :h], i[:, h:]
        if top:
            better = b > a
        else:
            better = (b > a) | ((b == a) & (ib < ia))
        w = jnp.where(better, b, a)
        iw = jnp.where(better, ib, ia)
        l = jnp.where(better, a, b)
        il = jnp.where(better, ia, ib)
        return (extract(w, iw, k, False)
                + extract(l, il, k // 2, False))

    cands = extract(score, iota, K, True)
    cv = jnp.stack([c[0] for c in cands], axis=1)
    ci_all = jnp.stack([c[1] for c in cands], axis=1)
    for k in range(K):
        m = jnp.max(cv, axis=1, keepdims=True)
        ci = jnp.min(jnp.where(cv == m, ci_all, N), axis=1)
        valid = m[:, 0] > neg_inf
        map_ref[k, :] = jnp.where(valid, ci, 0)
        gidx_ref[k, :] = jnp.where(valid, ci, PAD_ROW)
        cv = jnp.where(ci_all == ci[:, None], neg_inf, cv)


def _topk_tc(qpad, rt):
    grid = (Q // BQ,)
    return pl.pallas_call(
        _topk_body,
        grid=grid,
        in_specs=[
            pl.BlockSpec((BQ, 8), lambda i: (i, 0)),
            pl.BlockSpec((8, N), lambda i: (0, 0)),
        ],
        out_specs=[
            pl.BlockSpec((KPAD, BQ), lambda i: (0, i)),
            pl.BlockSpec((KPAD, BQ), lambda i: (0, i)),
        ],
        out_shape=[
            jax.ShapeDtypeStruct((KPAD, Q), jnp.int32),
            jax.ShapeDtypeStruct((KPAD, Q), jnp.int32),
        ],
        compiler_params=pltpu.CompilerParams(
            dimension_semantics=("arbitrary",),
        ),
    )(qpad, rt)


def _gather_body(table_hbm, idx_hbm, out_hbm, idx_v, rows_v, sem):
    wid = lax.axis_index("s") * SC_CORES + lax.axis_index("c")
    base = wid * B_PER_W
    pltpu.sync_copy(idx_hbm.at[wid], idx_v)
    for j in range(NCHUNK):
        pltpu.async_copy(
            table_hbm.at[idx_v.at[j]],
            rows_v.at[pl.ds(j * CHUNK, CHUNK)],
            sem,
        ).wait()
    pltpu.sync_copy(rows_v, out_hbm.at[pl.ds(base, B_PER_W)])


@functools.lru_cache(maxsize=1)
def _gather_sc():
    return pl.kernel(
        _gather_body,
        out_type=jax.ShapeDtypeStruct((B_TOTAL, 16), jnp.float32),
        mesh=plsc.VectorSubcoreMesh(
            core_axis_name="c", subcore_axis_name="s",
            num_cores=SC_CORES, num_subcores=SC_SUBCORES,
        ),
        scratch_types=[
            pltpu.VMEM((NCHUNK, CHUNK), jnp.int32),
            pltpu.VMEM((B_PER_W, 16), jnp.float32),
            pltpu.SemaphoreType.DMA,
        ],
        compiler_params=pltpu.CompilerParams(use_tc_tiling_on_sc=False),
    )


def kernel(x, p_grid):
    refs = x[0]
    q = jnp.reshape(p_grid, (Q, 3))
    qpad = jnp.pad(q, ((0, 0), (0, 5)))
    rt = jnp.pad(refs, ((0, 0), (0, 5))).T

    map16, gidx16 = _topk_tc(qpad, rt)
    mapping = map16[:K].T
    gidx3d = jnp.reshape(gidx16[:K].T, (NW, NCHUNK, CHUNK))

    table = jnp.zeros((TROWS, 16), jnp.float32).at[:N, :3].set(refs)
    rows = _gather_sc()(table, gidx3d)
    outputs = jnp.reshape(rows[:, :3], (1, Q, K, 3))
    return jnp.reshape(mapping, (1, Q, K)), outputs


# speedup vs baseline: 1.0457x; 1.0199x over previous
import functools

import jax
import jax.numpy as jnp
from jax import lax
from jax.experimental import pallas as pl
from jax.experimental.pallas import tpu as pltpu
from jax.experimental.pallas import tpu_sc as plsc

Q = 4096
N = 8192
K = 10
R2 = 0.25 * 0.25
BQ = 128
KPAD = 16
PAD_ROW = N
TROWS = N + 8

SC_CORES = 2
SC_SUBCORES = 16
NW = SC_CORES * SC_SUBCORES
B_TOTAL = Q * K
B_PER_W = B_TOTAL // NW
CHUNK = 128
NCHUNK = B_PER_W // CHUNK


def _topk_body(q_ref, rt_ref, map_ref, gidx_ref):
    q = q_ref[...]
    rt = rt_ref[...]
    qb = q.astype(jnp.bfloat16)
    rb = rt.astype(jnp.bfloat16)
    cross2 = jnp.dot(qb + qb, rb, preferred_element_type=jnp.float32)
    qn = ((q[:, 0:1] * q[:, 0:1] + q[:, 1:2] * q[:, 1:2])
          + q[:, 2:3] * q[:, 2:3])
    rn = ((rt[0:1, :] * rt[0:1, :] + rt[1:2, :] * rt[1:2, :])
          + rt[2:3, :] * rt[2:3, :])
    d2 = (qn + rn) - cross2
    d2 = jnp.maximum(d2, 0.0)
    neg_inf = jnp.float32(-jnp.inf)
    score = jnp.where(d2 <= R2, -d2, neg_inf)
    iota = lax.broadcasted_iota(jnp.int32, score.shape, 1)

    def direct(v, i, k):
        cands = []
        for _ in range(k):
            m = jnp.max(v, axis=1, keepdims=True)
            ci = jnp.min(jnp.where(v == m, i, N), axis=1)
            cands.append((m[:, 0], ci))
            v = jnp.where(i == ci[:, None], neg_inf, v)
        return cands

    def extract(v, i, k, top):
        if k == 0:
            return []
        h = v.shape[1] // 2
        if k == 1 or h < 1024:
            return direct(v, i, k)
        a, b = v[:, :h], v[:, h:]
        ia, ib = i[:, :h], i[:, h:]
        if top:
            better = b > a
        else:
            better = (b > a) | ((b == a) & (ib < ia))
        w = jnp.where(better, b, a)
        iw = jnp.where(better, ib, ia)
        l = jnp.where(better, a, b)
        il = jnp.where(better, ia, ib)
        return (extract(w, iw, k, False)
                + extract(l, il, k // 2, False))

    cands = extract(score, iota, K, True)
    cv = jnp.stack([c[0] for c in cands], axis=1)
    ci_all = jnp.stack([c[1] for c in cands], axis=1)
    for k in range(K):
        m = jnp.max(cv, axis=1, keepdims=True)
        ci = jnp.min(jnp.where(cv == m, ci_all, N), axis=1)
        valid = m[:, 0] > neg_inf
        map_ref[k, :] = jnp.where(valid, ci, 0)
        gidx_ref[k, :] = jnp.where(valid, ci, PAD_ROW)
        cv = jnp.where(ci_all == ci[:, None], neg_inf, cv)


def _topk_tc(qpad, rt):
    grid = (Q // BQ,)
    return pl.pallas_call(
        _topk_body,
        grid=grid,
        in_specs=[
            pl.BlockSpec((BQ, 8), lambda i: (i, 0)),
            pl.BlockSpec((8, N), lambda i: (0, 0)),
        ],
        out_specs=[
            pl.BlockSpec((KPAD, BQ), lambda i: (0, i)),
            pl.BlockSpec((KPAD, BQ), lambda i: (0, i)),
        ],
        out_shape=[
            jax.ShapeDtypeStruct((KPAD, Q), jnp.int32),
            jax.ShapeDtypeStruct((KPAD, Q), jnp.int32),
        ],
        compiler_params=pltpu.CompilerParams(
            dimension_semantics=("arbitrary",),
        ),
    )(qpad, rt)


def _gather_body(table_hbm, idx_hbm, out_hbm, idx_v, rows_v, sem):
    wid = lax.axis_index("s") * SC_CORES + lax.axis_index("c")
    base = wid * B_PER_W
    pltpu.sync_copy(idx_hbm.at[wid], idx_v)
    for j in range(NCHUNK):
        pltpu.async_copy(
            table_hbm.at[idx_v.at[j]],
            rows_v.at[pl.ds(j * CHUNK, CHUNK)],
            sem,
        ).wait()
    pltpu.sync_copy(rows_v, out_hbm.at[pl.ds(base, B_PER_W)])


@functools.lru_cache(maxsize=1)
def _gather_sc():
    return pl.kernel(
        _gather_body,
        out_type=jax.ShapeDtypeStruct((B_TOTAL, 16), jnp.float32),
        mesh=plsc.VectorSubcoreMesh(
            core_axis_name="c", subcore_axis_name="s",
            num_cores=SC_CORES, num_subcores=SC_SUBCORES,
        ),
        scratch_types=[
            pltpu.VMEM((NCHUNK, CHUNK), jnp.int32),
            pltpu.VMEM((B_PER_W, 16), jnp.float32),
            pltpu.SemaphoreType.DMA,
        ],
        compiler_params=pltpu.CompilerParams(use_tc_tiling_on_sc=False),
    )


def kernel(x, p_grid):
    refs = x[0]
    q = jnp.reshape(p_grid, (Q, 3))
    qpad = jnp.pad(q, ((0, 0), (0, 5)))
    rt = jnp.pad(refs, ((0, 0), (0, 5))).T

    map16, gidx16 = _topk_tc(qpad, rt)
    mapping = map16[:K].T
    gidx3d = jnp.reshape(gidx16[:K].T, (NW, NCHUNK, CHUNK))

    table = jnp.zeros((TROWS, 16), jnp.float32).at[:N, :3].set(refs)
    rows = _gather_sc()(table, gidx3d)
    outputs = jnp.reshape(rows[:, :3], (1, Q, K, 3))
    return jnp.reshape(mapping, (1, Q, K)), outputs


# speedup vs baseline: 1.0610x; 1.0146x over previous
import functools

import jax
import jax.numpy as jnp
from jax import lax
from jax.experimental import pallas as pl
from jax.experimental.pallas import tpu as pltpu
from jax.experimental.pallas import tpu_sc as plsc

Q = 4096
N = 8192
K = 10
R2 = 0.25 * 0.25
BQ = 128
KPAD = 16
PAD_ROW = N
TROWS = N + 8

SC_CORES = 2
SC_SUBCORES = 16
NW = SC_CORES * SC_SUBCORES
B_TOTAL = Q * K
B_PER_W = B_TOTAL // NW
CHUNK = 128
NCHUNK = B_PER_W // CHUNK


def _topk_body(q_ref, rt_ref, map_ref, gidx_ref):
    q = q_ref[...]
    rt = rt_ref[...]
    qb = q.astype(jnp.bfloat16)
    rb = rt.astype(jnp.bfloat16)
    cross2 = jnp.dot(qb + qb, rb, preferred_element_type=jnp.float32)
    qn = ((q[:, 0:1] * q[:, 0:1] + q[:, 1:2] * q[:, 1:2])
          + q[:, 2:3] * q[:, 2:3])
    rn = ((rt[0:1, :] * rt[0:1, :] + rt[1:2, :] * rt[1:2, :])
          + rt[2:3, :] * rt[2:3, :])
    d2 = (qn + rn) - cross2
    d2 = jnp.maximum(d2, 0.0)
    pos_inf = jnp.float32(jnp.inf)
    score = jnp.where(d2 <= R2, d2, pos_inf)
    iota = lax.broadcasted_iota(jnp.int32, score.shape, 1)

    def direct(v, i, k):
        cands = []
        for _ in range(k):
            m = jnp.min(v, axis=1, keepdims=True)
            ci = jnp.min(jnp.where(v == m, i, N), axis=1)
            cands.append((m[:, 0], ci))
            v = jnp.where(i == ci[:, None], pos_inf, v)
        return cands

    def extract(v, i, k, top):
        if k == 0:
            return []
        h = v.shape[1] // 2
        if k == 1 or h < 1024:
            return direct(v, i, k)
        a, b = v[:, :h], v[:, h:]
        ia, ib = i[:, :h], i[:, h:]
        if top:
            better = b < a
        else:
            better = (b < a) | ((b == a) & (ib < ia))
        w = jnp.where(better, b, a)
        iw = jnp.where(better, ib, ia)
        l = jnp.where(better, a, b)
        il = jnp.where(better, ia, ib)
        return (extract(w, iw, k, False)
                + extract(l, il, k // 2, False))

    cands = extract(score, iota, K, True)
    cv = jnp.stack([c[0] for c in cands], axis=1)
    ci_all = jnp.stack([c[1] for c in cands], axis=1)
    for k in range(K):
        m = jnp.min(cv, axis=1, keepdims=True)
        ci = jnp.min(jnp.where(cv == m, ci_all, N), axis=1)
        valid = m[:, 0] < pos_inf
        map_ref[k, :] = jnp.where(valid, ci, 0)
        gidx_ref[k, :] = jnp.where(valid, ci, PAD_ROW)
        cv = jnp.where(ci_all == ci[:, None], pos_inf, cv)


def _topk_tc(qpad, rt):
    grid = (Q // BQ,)
    return pl.pallas_call(
        _topk_body,
        grid=grid,
        in_specs=[
            pl.BlockSpec((BQ, 8), lambda i: (i, 0)),
            pl.BlockSpec((8, N), lambda i: (0, 0)),
        ],
        out_specs=[
            pl.BlockSpec((KPAD, BQ), lambda i: (0, i)),
            pl.BlockSpec((KPAD, BQ), lambda i: (0, i)),
        ],
        out_shape=[
            jax.ShapeDtypeStruct((KPAD, Q), jnp.int32),
            jax.ShapeDtypeStruct((KPAD, Q), jnp.int32),
        ],
        compiler_params=pltpu.CompilerParams(
            dimension_semantics=("arbitrary",),
        ),
    )(qpad, rt)


def _gather_body(table_hbm, idx_hbm, out_hbm, idx_v, rows_v, sem):
    wid = lax.axis_index("s") * SC_CORES + lax.axis_index("c")
    base = wid * B_PER_W
    pltpu.sync_copy(idx_hbm.at[wid], idx_v)
    for j in range(NCHUNK):
        pltpu.async_copy(
            table_hbm.at[idx_v.at[j]],
            rows_v.at[pl.ds(j * CHUNK, CHUNK)],
            sem,
        ).wait()
    pltpu.sync_copy(rows_v, out_hbm.at[pl.ds(base, B_PER_W)])


@functools.lru_cache(maxsize=1)
def _gather_sc():
    return pl.kernel(
        _gather_body,
        out_type=jax.ShapeDtypeStruct((B_TOTAL, 16), jnp.float32),
        mesh=plsc.VectorSubcoreMesh(
            core_axis_name="c", subcore_axis_name="s",
            num_cores=SC_CORES, num_subcores=SC_SUBCORES,
        ),
        scratch_types=[
            pltpu.VMEM((NCHUNK, CHUNK), jnp.int32),
            pltpu.VMEM((B_PER_W, 16), jnp.float32),
            pltpu.SemaphoreType.DMA,
        ],
        compiler_params=pltpu.CompilerParams(use_tc_tiling_on_sc=False),
    )


def kernel(x, p_grid):
    refs = x[0]
    q = jnp.reshape(p_grid, (Q, 3))
    qpad = jnp.pad(q, ((0, 0), (0, 5)))
    rt = jnp.pad(refs, ((0, 0), (0, 5))).T

    map16, gidx16 = _topk_tc(qpad, rt)
    mapping = map16[:K].T
    gidx3d = jnp.reshape(gidx16[:K].T, (NW, NCHUNK, CHUNK))

    table = jnp.zeros((TROWS, 16), jnp.float32).at[:N, :3].set(refs)
    rows = _gather_sc()(table, gidx3d)
    outputs = jnp.reshape(rows[:, :3], (1, Q, K, 3))
    return jnp.reshape(mapping, (1, Q, K)), outputs
